# Initial kernel scaffold; baseline (speedup 1.0000x reference)
import jax
import jax.numpy as jnp
from jax.experimental import pallas as pl


def kernel(x, weight, fc_table, bias):
    raise NotImplementedError("write your pallas kernel here")


# speedup vs baseline: 1.4057x; 1.4057x over previous
import functools

import jax
import jax.numpy as jnp
from jax import lax
from jax.experimental import pallas as pl
from jax.experimental.pallas import tpu as pltpu
from jax.experimental.pallas import tpu_sc as plsc

B = 16384
F = 26
FIELD = 40000
TOTAL_VOCAB = F * FIELD
NC = 2
NS = 16
L = 16
NW = NC * NS
BPW = B // NW
NCHUNK = BPW // L
PER_W = F * BPW


def _sc_body(x_hbm, w_hbm, table_hbm, bias_hbm, out_hbm,
             idx_v, val_v, w_v, out_v, bias_v, sem):
    c = lax.axis_index("c")
    s = lax.axis_index("s")
    wid = s * NC + c

    pltpu.sync_copy(x_hbm.at[wid], idx_v)
    pltpu.sync_copy(w_hbm.at[wid], w_v)
    pltpu.sync_copy(bias_hbm, bias_v)

    for f in range(1, F):
        off = jnp.int32(f * FIELD)

        def _add(i, _, f=f, off=off):
            sl = pl.ds(f * BPW + i * L, L)
            idx_v[sl] = idx_v[sl] + off
            return _

        lax.fori_loop(0, NCHUNK, _add, 0, unroll=4)

    pltpu.async_copy(table_hbm.at[idx_v], val_v, sem).wait()

    def _reduce(i, _):
        sl0 = pl.ds(i * L, L)
        acc = bias_v[...] + val_v[sl0] * w_v[sl0]
        for f in range(1, F):
            sl = pl.ds(f * BPW + i * L, L)
            acc = acc + val_v[sl] * w_v[sl]
        out_v[sl0] = acc
        return _

    lax.fori_loop(0, NCHUNK, _reduce, 0)

    pltpu.sync_copy(out_v, out_hbm.at[pl.ds(wid * BPW, BPW)])


@jax.jit
def kernel(x, weight, fc_table, bias):
    x_t = x.astype(jnp.int32).reshape(NW, BPW, F).transpose(0, 2, 1).reshape(NW, PER_W)
    w_t = weight.reshape(NW, BPW, F).transpose(0, 2, 1).reshape(NW, PER_W)
    table_flat = fc_table.reshape(TOTAL_VOCAB)
    bias16 = jnp.broadcast_to(bias.reshape(1), (L,))

    mesh = plsc.VectorSubcoreMesh(core_axis_name="c", subcore_axis_name="s")
    out = pl.kernel(
        _sc_body,
        mesh=mesh,
        out_type=jax.ShapeDtypeStruct((B,), jnp.float32),
        scratch_types=[
            pltpu.VMEM((PER_W,), jnp.int32),
            pltpu.VMEM((PER_W,), jnp.float32),
            pltpu.VMEM((PER_W,), jnp.float32),
            pltpu.VMEM((BPW,), jnp.float32),
            pltpu.VMEM((L,), jnp.float32),
            pltpu.SemaphoreType.DMA,
        ],
    )(x_t, w_t, table_flat, bias16)
    return out.reshape(B, 1)
